# baseline (speedup 1.0000x reference)
import dataclasses

import jax
import jax.numpy as jnp
from jax import lax
from jax.experimental import pallas as pl
from jax.experimental.pallas import tpu as pltpu
from jax.experimental.pallas import tpu_sc as plsc

P = 0.2
CH_PER_BLOCK = 24


def _copy_kernel(x_ref, o_ref):
    o_ref[...] = x_ref[...]


def _sc_target_body(r_hbm, xs_hbm, o_hbm, r_s, xs_s, t_s, sem):
    B, C = r_hbm.shape[0], xs_hbm.shape[0]
    cid = lax.axis_index("core")

    @pl.when(cid == 0)
    def _():
        pltpu.make_async_copy(r_hbm, r_s, sem).start()
        pltpu.make_async_copy(r_hbm, r_s, sem).wait()
        pltpu.make_async_copy(xs_hbm, xs_s, sem).start()
        pltpu.make_async_copy(xs_hbm, xs_s, sem).wait()

        def per_sample(b, _):
            r0 = r_s[b, 0]
            r1 = r_s[b, 1]

            def step(_, lohi):
                lo, hi = lohi
                mid = (lo + hi) // 2
                pred = r0 > xs_s[mid]
                return jnp.where(pred, mid + 1, lo), jnp.where(pred, hi, mid)

            lo, _ = jax.lax.fori_loop(
                0, 8, step, (jnp.int32(0), jnp.int32(C)))
            t_s[0, b] = jnp.where(r1 < P, lo, jnp.int32(-1))
            return 0

        jax.lax.fori_loop(0, B, per_sample, 0)
        pltpu.make_async_copy(t_s, o_hbm, sem).start()
        pltpu.make_async_copy(t_s, o_hbm, sem).wait()


def _scatter_kernel(t_ref, a_ref, o_ref, zbuf, sem):
    B, C, H, W = o_ref.shape
    zbuf[...] = jnp.zeros((H, W), jnp.float32)

    def body(b, _):
        tgt = t_ref[0, b]

        @pl.when(tgt >= 0)
        def _():
            cp = pltpu.make_async_copy(zbuf, o_ref.at[b, tgt], sem)
            cp.start()
            cp.wait()

        return 0

    jax.lax.fori_loop(0, B, body, 0)


def kernel(tensor, r):
    B, C, H, W = tensor.shape
    xs = jnp.linspace(1.0 / C, 1.0, C).astype(jnp.float32)

    CB = CH_PER_BLOCK
    copied = pl.pallas_call(
        _copy_kernel,
        grid=(B, C // CB),
        in_specs=[pl.BlockSpec((1, CB, H, W), lambda b, j: (b, j, 0, 0))],
        out_specs=pl.BlockSpec((1, CB, H, W), lambda b, j: (b, j, 0, 0)),
        out_shape=jax.ShapeDtypeStruct((B, C, H, W), jnp.float32),
    )(tensor)

    mesh = plsc.ScalarSubcoreMesh(axis_name="core", num_cores=2)
    cp = pltpu.CompilerParams()
    if "needs_layout_passes" in pltpu.CompilerParams.__dataclass_fields__:
        cp = dataclasses.replace(cp, needs_layout_passes=False)
    targets = pl.kernel(
        _sc_target_body,
        out_type=jax.ShapeDtypeStruct((1, B), jnp.int32),
        mesh=mesh,
        compiler_params=cp,
        scratch_types=[
            pltpu.SMEM((B, 2), jnp.float32),
            pltpu.SMEM((C,), jnp.float32),
            pltpu.SMEM((1, B), jnp.int32),
            pltpu.SemaphoreType.DMA,
        ],
    )(r, xs)

    out = pl.pallas_call(
        _scatter_kernel,
        in_specs=[
            pl.BlockSpec(memory_space=pltpu.SMEM),
            pl.BlockSpec(memory_space=pltpu.MemorySpace.HBM),
        ],
        out_specs=pl.BlockSpec(memory_space=pltpu.MemorySpace.HBM),
        out_shape=jax.ShapeDtypeStruct((B, C, H, W), jnp.float32),
        scratch_shapes=[
            pltpu.VMEM((H, W), jnp.float32),
            pltpu.SemaphoreType.DMA,
        ],
        input_output_aliases={1: 0},
    )(targets, copied)
    return out


# speedup vs baseline: 1.0025x; 1.0025x over previous
import dataclasses

import jax
import jax.numpy as jnp
from jax import lax
from jax.experimental import pallas as pl
from jax.experimental.pallas import tpu as pltpu
from jax.experimental.pallas import tpu_sc as plsc

P = 0.2
CH_PER_BLOCK = 24


def _copy_kernel(x_ref, o_ref):
    o_ref[...] = x_ref[...]


def _sc_target_body(rT_hbm, xs_hbm, o_hbm, r_v, xs_v, t_v, sem):
    C = xs_hbm.shape[0]
    B = o_hbm.shape[1]
    wid = lax.axis_index("s") * 2 + lax.axis_index("c")

    @pl.when(wid == 0)
    def _():
        pltpu.make_async_copy(rT_hbm, r_v, sem).start()
        pltpu.make_async_copy(rT_hbm, r_v, sem).wait()
        pltpu.make_async_copy(xs_hbm, xs_v, sem).start()
        pltpu.make_async_copy(xs_hbm, xs_v, sem).wait()

        lane = lax.iota(jnp.int32, 16)
        r0 = r_v[0, :]
        r1 = r_v[1, :]
        tgt = jnp.full((16,), -1, jnp.int32)
        for b in range(B):
            self = (lane == b).astype(jnp.float32)
            r0b = jnp.sum(r0 * self, axis=0)
            r1b = jnp.sum(r1 * self, axis=0)
            cnt = jnp.int32(0)
            for k in range(C // 16):
                cnt = cnt + jnp.sum(
                    (r0b > xs_v[pl.ds(k * 16, 16)]).astype(jnp.int32), axis=0)
            tb = jnp.where(r1b < P, cnt, jnp.int32(-1))
            tgt = jnp.where(lane == b, tb, tgt)
        t_v[...] = tgt
        pltpu.make_async_copy(t_v, o_hbm.at[0], sem).start()
        pltpu.make_async_copy(t_v, o_hbm.at[0], sem).wait()


def _scatter_kernel(t_ref, a_ref, o_ref, zbuf, sem):
    B, C, H, W = o_ref.shape
    zbuf[...] = jnp.zeros((H, W), jnp.float32)

    def body(b, _):
        tgt = t_ref[0, b]

        @pl.when(tgt >= 0)
        def _():
            cp = pltpu.make_async_copy(zbuf, o_ref.at[b, tgt], sem)
            cp.start()
            cp.wait()

        return 0

    jax.lax.fori_loop(0, B, body, 0)


def kernel(tensor, r):
    B, C, H, W = tensor.shape
    xs = jnp.linspace(1.0 / C, 1.0, C).astype(jnp.float32)
    rT = r.T.astype(jnp.float32)

    CB = CH_PER_BLOCK
    copied = pl.pallas_call(
        _copy_kernel,
        grid=(B, C // CB),
        in_specs=[pl.BlockSpec((1, CB, H, W), lambda b, j: (b, j, 0, 0))],
        out_specs=pl.BlockSpec((1, CB, H, W), lambda b, j: (b, j, 0, 0)),
        out_shape=jax.ShapeDtypeStruct((B, C, H, W), jnp.float32),
    )(tensor)

    mesh = plsc.VectorSubcoreMesh(core_axis_name="c", subcore_axis_name="s")
    cp = pltpu.CompilerParams()
    if "needs_layout_passes" in pltpu.CompilerParams.__dataclass_fields__:
        cp = dataclasses.replace(cp, needs_layout_passes=False)
    targets = pl.kernel(
        _sc_target_body,
        out_type=jax.ShapeDtypeStruct((1, B), jnp.int32),
        mesh=mesh,
        compiler_params=cp,
        scratch_types=[
            pltpu.VMEM((2, 16), jnp.float32),
            pltpu.VMEM((C,), jnp.float32),
            pltpu.VMEM((16,), jnp.int32),
            pltpu.SemaphoreType.DMA,
        ],
    )(rT, xs)

    out = pl.pallas_call(
        _scatter_kernel,
        in_specs=[
            pl.BlockSpec(memory_space=pltpu.SMEM),
            pl.BlockSpec(memory_space=pltpu.MemorySpace.HBM),
        ],
        out_specs=pl.BlockSpec(memory_space=pltpu.MemorySpace.HBM),
        out_shape=jax.ShapeDtypeStruct((B, C, H, W), jnp.float32),
        scratch_shapes=[
            pltpu.VMEM((H, W), jnp.float32),
            pltpu.SemaphoreType.DMA,
        ],
        input_output_aliases={1: 0},
    )(targets, copied)
    return out
